# initial kernel scaffold (unmeasured)
import jax
import jax.numpy as jnp
from jax import lax
from jax.experimental import pallas as pl
from jax.experimental.pallas import tpu as pltpu


def kernel(x, dest):
    t, d = x.shape
    dest2 = dest.reshape(1, t).astype(jnp.int32)

    def body(x_ref, dest_ref, out_ref, xs, px, pd, send_sems, recv_sems):
        p = lax.axis_index("y")
        mx = lax.axis_index("x")
        mz = lax.axis_index("z")
        peer = (mx, 1 - p, mz)

        xs[...] = x_ref[...].astype(jnp.bfloat16)

        bar = pltpu.get_barrier_semaphore()
        pl.semaphore_signal(
            bar, inc=1, device_id=peer, device_id_type=pl.DeviceIdType.MESH
        )
        pl.semaphore_wait(bar, 1)

        rdma_x = pltpu.make_async_remote_copy(
            src_ref=xs,
            dst_ref=px,
            send_sem=send_sems.at[0],
            recv_sem=recv_sems.at[0],
            device_id=peer,
            device_id_type=pl.DeviceIdType.MESH,
        )
        rdma_d = pltpu.make_async_remote_copy(
            src_ref=dest_ref,
            dst_ref=pd,
            send_sem=send_sems.at[1],
            recv_sem=recv_sems.at[1],
            device_id=peer,
            device_id_type=pl.DeviceIdType.MESH,
        )
        rdma_x.start()
        rdma_d.start()
        rdma_d.wait()
        rdma_x.wait()

        md_self = dest_ref[...] == p
        md_peer = pd[...] == p
        mself = md_self.astype(jnp.float32)
        mpeer = md_peer.astype(jnp.float32)

        tri = (
            lax.broadcasted_iota(jnp.int32, (t, t), 0)
            <= lax.broadcasted_iota(jnp.int32, (t, t), 1)
        ).astype(jnp.float32)
        c_self = jnp.dot(mself, tri, preferred_element_type=jnp.float32) - mself
        c_peer = jnp.dot(mpeer, tri, preferred_element_type=jnp.float32) - mpeer

        n_self = jnp.sum(mself)
        n_peer = jnp.sum(mpeer)
        i_am_lo = (p == 0).astype(jnp.float32)
        off_self = (1.0 - i_am_lo) * n_peer
        off_peer = i_am_lo * n_self

        rank_self = c_self + off_self
        rank_peer = c_peer + off_peer

        i_iota = lax.broadcasted_iota(jnp.float32, (t, t), 0)
        p_self = ((i_iota == rank_self) & md_self).astype(jnp.bfloat16)
        p_peer = ((i_iota == rank_peer) & md_peer).astype(jnp.bfloat16)

        acc = jnp.dot(p_self, xs[...], preferred_element_type=jnp.float32)
        acc = acc + jnp.dot(p_peer, px[...], preferred_element_type=jnp.float32)
        out_ref[...] = acc.astype(jnp.bfloat16)

    return pl.pallas_call(
        body,
        out_shape=jax.ShapeDtypeStruct((t, d), jnp.bfloat16),
        in_specs=[
            pl.BlockSpec(memory_space=pltpu.VMEM),
            pl.BlockSpec(memory_space=pltpu.VMEM),
        ],
        out_specs=pl.BlockSpec(memory_space=pltpu.VMEM),
        scratch_shapes=[
            pltpu.VMEM((t, d), jnp.bfloat16),
            pltpu.VMEM((t, d), jnp.bfloat16),
            pltpu.VMEM((1, t), jnp.int32),
            pltpu.SemaphoreType.DMA((2,)),
            pltpu.SemaphoreType.DMA((2,)),
        ],
        compiler_params=pltpu.CompilerParams(collective_id=0),
    )(x, dest2)


# baseline (device time: 21162 ns/iter reference)
import jax
import jax.numpy as jnp
from jax import lax
from jax.experimental import pallas as pl
from jax.experimental.pallas import tpu as pltpu


def kernel(x, dest):
    t, d = x.shape
    dest2 = dest.reshape(1, t).astype(jnp.int32)

    def body(x_ref, dest_ref, out_ref, xs, px, pd, send_sems, recv_sems):
        p = lax.axis_index("y")
        mx = lax.axis_index("x")
        mz = lax.axis_index("z")
        peer = (mx, 1 - p, mz)

        xs[...] = x_ref[...].astype(jnp.bfloat16)

        bar = pltpu.get_barrier_semaphore()
        pl.semaphore_signal(
            bar, inc=1, device_id=peer, device_id_type=pl.DeviceIdType.MESH
        )
        pl.semaphore_wait(bar, 1)

        rdma_x = pltpu.make_async_remote_copy(
            src_ref=xs,
            dst_ref=px,
            send_sem=send_sems.at[0],
            recv_sem=recv_sems.at[0],
            device_id=peer,
            device_id_type=pl.DeviceIdType.MESH,
        )
        rdma_d = pltpu.make_async_remote_copy(
            src_ref=dest_ref,
            dst_ref=pd,
            send_sem=send_sems.at[1],
            recv_sem=recv_sems.at[1],
            device_id=peer,
            device_id_type=pl.DeviceIdType.MESH,
        )
        rdma_x.start()
        rdma_d.start()
        rdma_d.wait()
        rdma_x.wait()

        md_self = dest_ref[...] == p
        md_peer = pd[...] == p
        mself = md_self.astype(jnp.float32)
        mpeer = md_peer.astype(jnp.float32)

        tri = (
            lax.broadcasted_iota(jnp.int32, (t, t), 0)
            <= lax.broadcasted_iota(jnp.int32, (t, t), 1)
        ).astype(jnp.float32)
        c_self = jnp.dot(mself, tri, preferred_element_type=jnp.float32) - mself
        c_peer = jnp.dot(mpeer, tri, preferred_element_type=jnp.float32) - mpeer

        n_self = jnp.sum(mself)
        n_peer = jnp.sum(mpeer)
        i_am_lo = (p == 0).astype(jnp.float32)
        off_self = (1.0 - i_am_lo) * n_peer
        off_peer = i_am_lo * n_self

        rank_self = c_self + off_self
        rank_peer = c_peer + off_peer

        i_iota = lax.broadcasted_iota(jnp.int32, (t, t), 0)
        p_self = ((i_iota == rank_self.astype(jnp.int32)) & md_self).astype(
            jnp.bfloat16
        )
        p_peer = ((i_iota == rank_peer.astype(jnp.int32)) & md_peer).astype(
            jnp.bfloat16
        )

        acc = jnp.dot(p_self, xs[...], preferred_element_type=jnp.float32)
        acc = acc + jnp.dot(p_peer, px[...], preferred_element_type=jnp.float32)
        out_ref[...] = acc.astype(jnp.bfloat16)

    return pl.pallas_call(
        body,
        out_shape=jax.ShapeDtypeStruct((t, d), jnp.bfloat16),
        in_specs=[
            pl.BlockSpec(memory_space=pltpu.VMEM),
            pl.BlockSpec(memory_space=pltpu.VMEM),
        ],
        out_specs=pl.BlockSpec(memory_space=pltpu.VMEM),
        scratch_shapes=[
            pltpu.VMEM((t, d), jnp.bfloat16),
            pltpu.VMEM((t, d), jnp.bfloat16),
            pltpu.VMEM((1, t), jnp.int32),
            pltpu.SemaphoreType.DMA((2,)),
            pltpu.SemaphoreType.DMA((2,)),
        ],
        compiler_params=pltpu.CompilerParams(collective_id=0),
    )(x, dest2)


# device time: 18658 ns/iter; 1.1342x vs baseline; 1.1342x over previous
import jax
import jax.numpy as jnp
from jax import lax
from jax.experimental import pallas as pl
from jax.experimental.pallas import tpu as pltpu

NC = 8


def kernel(x, dest):
    t, d = x.shape
    c = t // NC
    dest2 = dest.reshape(1, t).astype(jnp.int32)

    def body(x_ref, dest_ref, out_ref, xs, px, pd, send_sems, recv_sems):
        p = lax.axis_index("y")
        mx = lax.axis_index("x")
        mz = lax.axis_index("z")
        peer = (mx, 1 - p, mz)

        xs[...] = x_ref[...].astype(jnp.bfloat16)

        bar = pltpu.get_barrier_semaphore()
        pl.semaphore_signal(
            bar, inc=1, device_id=peer, device_id_type=pl.DeviceIdType.MESH
        )
        pl.semaphore_wait(bar, 1)

        rdma_d = pltpu.make_async_remote_copy(
            src_ref=dest_ref,
            dst_ref=pd,
            send_sem=send_sems.at[NC],
            recv_sem=recv_sems.at[NC],
            device_id=peer,
            device_id_type=pl.DeviceIdType.MESH,
        )
        rdma_d.start()
        rdma_x = []
        for k in range(NC):
            r = pltpu.make_async_remote_copy(
                src_ref=xs.at[pl.ds(k * c, c)],
                dst_ref=px.at[pl.ds(k * c, c)],
                send_sem=send_sems.at[k],
                recv_sem=recv_sems.at[k],
                device_id=peer,
                device_id_type=pl.DeviceIdType.MESH,
            )
            r.start()
            rdma_x.append(r)

        md_self = dest_ref[...] == p
        mself = md_self.astype(jnp.float32)
        tri = (
            lax.broadcasted_iota(jnp.int32, (t, t), 0)
            <= lax.broadcasted_iota(jnp.int32, (t, t), 1)
        ).astype(jnp.float32)
        c_self = jnp.dot(mself, tri, preferred_element_type=jnp.float32) - mself
        n_self = jnp.sum(mself)
        i_iota = lax.broadcasted_iota(jnp.int32, (t, t), 0)

        rdma_d.wait()
        md_peer = pd[...] == p
        mpeer = md_peer.astype(jnp.float32)
        c_peer = jnp.dot(mpeer, tri, preferred_element_type=jnp.float32) - mpeer
        n_peer = jnp.sum(mpeer)

        i_am_lo = (p == 0).astype(jnp.float32)
        off_self = (1.0 - i_am_lo) * n_peer
        off_peer = i_am_lo * n_self
        rank_self = (c_self + off_self).astype(jnp.int32)
        rank_peer = (c_peer + off_peer).astype(jnp.int32)

        p_self = ((i_iota == rank_self) & md_self).astype(jnp.bfloat16)
        p_peer = ((i_iota == rank_peer) & md_peer).astype(jnp.bfloat16)
        acc = jnp.dot(p_self, xs[...], preferred_element_type=jnp.float32)

        for k in range(NC):
            rdma_x[k].wait_recv()
            acc = acc + jnp.dot(
                p_peer[:, k * c : (k + 1) * c],
                px[pl.ds(k * c, c), :],
                preferred_element_type=jnp.float32,
            )
        out_ref[...] = acc.astype(jnp.bfloat16)

        for k in range(NC):
            rdma_x[k].wait_send()

    return pl.pallas_call(
        body,
        out_shape=jax.ShapeDtypeStruct((t, d), jnp.bfloat16),
        in_specs=[
            pl.BlockSpec(memory_space=pltpu.VMEM),
            pl.BlockSpec(memory_space=pltpu.VMEM),
        ],
        out_specs=pl.BlockSpec(memory_space=pltpu.VMEM),
        scratch_shapes=[
            pltpu.VMEM((t, d), jnp.bfloat16),
            pltpu.VMEM((t, d), jnp.bfloat16),
            pltpu.VMEM((1, t), jnp.int32),
            pltpu.SemaphoreType.DMA((NC + 1,)),
            pltpu.SemaphoreType.DMA((NC + 1,)),
        ],
        compiler_params=pltpu.CompilerParams(collective_id=0),
    )(x, dest2)


# device time: 14463 ns/iter; 1.4632x vs baseline; 1.2901x over previous
import jax
import jax.numpy as jnp
from jax import lax
from jax.experimental import pallas as pl
from jax.experimental.pallas import tpu as pltpu

NC = 16


def kernel(x, dest):
    t, d = x.shape
    c = t // NC
    dest2 = dest.reshape(1, t).astype(jnp.int32)

    def body(x_ref, dest_ref, out_ref, xs, xsend, px, send_sems, recv_sems):
        p = lax.axis_index("y")
        mx = lax.axis_index("x")
        mz = lax.axis_index("z")
        peer = (mx, 1 - p, mz)

        xs[...] = x_ref[...].astype(jnp.bfloat16)

        md = dest_ref[...] == p
        mdi = md.astype(jnp.int32)
        mdf = md.astype(jnp.float32)
        tri = (
            lax.broadcasted_iota(jnp.int32, (t, t), 0)
            <= lax.broadcasted_iota(jnp.int32, (t, t), 1)
        ).astype(jnp.float32)
        ck_incl = jnp.dot(mdf, tri, preferred_element_type=jnp.float32)
        ck_incl_i = ck_incl.astype(jnp.int32)
        j_vec = lax.broadcasted_iota(jnp.int32, (1, t), 1)
        rank_keep_local = ck_incl_i - mdi
        rank_send = j_vec - ck_incl_i + mdi

        n_self = jnp.sum(mdi)
        m = t - n_self

        bar = pltpu.get_barrier_semaphore()
        pl.semaphore_signal(
            bar, inc=1, device_id=peer, device_id_type=pl.DeviceIdType.MESH
        )
        pl.semaphore_wait(bar, 1)

        rdmas = []
        for k in range(NC):
            rk = pltpu.make_async_remote_copy(
                src_ref=xsend.at[pl.ds(k * c, c)],
                dst_ref=px.at[pl.ds(k * c, c)],
                send_sem=send_sems.at[k],
                recv_sem=recv_sems.at[k],
                device_id=peer,
                device_id_type=pl.DeviceIdType.MESH,
            )
            rdmas.append(rk)

            @pl.when(k * c < m)
            def _(k=k, rk=rk):
                i_io = lax.broadcasted_iota(jnp.int32, (c, t), 0) + k * c
                s_k = ((i_io == rank_send) & (~md)).astype(jnp.bfloat16)
                xsend[pl.ds(k * c, c), :] = jnp.dot(
                    s_k, xs[...], preferred_element_type=jnp.float32
                ).astype(jnp.bfloat16)
                rk.start()

        o_self = p * m
        rank_keep = rank_keep_local + o_self
        i_iota = lax.broadcasted_iota(jnp.int32, (t, t), 0)
        kmat = ((i_iota == rank_keep) & md).astype(jnp.bfloat16)
        own = jnp.dot(kmat, xs[...], preferred_element_type=jnp.float32).astype(
            jnp.bfloat16
        )

        for k in range(NC):
            @pl.when(k * c < m)
            def _(k=k):
                rdmas[k].wait_recv()

        o_in = (1 - p) * n_self
        row_i = lax.broadcasted_iota(jnp.int32, (t, 1), 0)
        pxm = jnp.where(row_i < m, px[...], jnp.array(0, jnp.bfloat16))
        out_ref[...] = own + pltpu.roll(pxm, o_in, 0)

        for k in range(NC):
            @pl.when(k * c < m)
            def _(k=k):
                rdmas[k].wait_send()

    return pl.pallas_call(
        body,
        out_shape=jax.ShapeDtypeStruct((t, d), jnp.bfloat16),
        in_specs=[
            pl.BlockSpec(memory_space=pltpu.VMEM),
            pl.BlockSpec(memory_space=pltpu.VMEM),
        ],
        out_specs=pl.BlockSpec(memory_space=pltpu.VMEM),
        scratch_shapes=[
            pltpu.VMEM((t, d), jnp.bfloat16),
            pltpu.VMEM((t, d), jnp.bfloat16),
            pltpu.VMEM((t, d), jnp.bfloat16),
            pltpu.SemaphoreType.DMA((NC + 1,)),
            pltpu.SemaphoreType.DMA((NC + 1,)),
        ],
        compiler_params=pltpu.CompilerParams(collective_id=0),
    )(x, dest2)
